# Initial kernel scaffold; baseline (speedup 1.0000x reference)
import jax
import jax.numpy as jnp
from jax.experimental import pallas as pl


def kernel(batch_original_indices, gnn_logits_batch, true_labels_batch_one_hot, gnn_embeddings_batch, batch_iter_num, current_epoch, atrain_overall_accuracy, u, prev_gnn_embeddings, class_centroids):
    raise NotImplementedError("write your pallas kernel here")


# speedup vs baseline: 1.5814x; 1.5814x over previous
import functools

import jax
import jax.numpy as jnp
from jax import lax
from jax.experimental import pallas as pl
from jax.experimental.pallas import tpu as pltpu
from jax.experimental.pallas import tpu_sc as plsc

EPS = 1e-07
NE = 100000
NC = 100
D = 256
B = 16384
L = 16
NW = 16

_mesh = plsc.VectorSubcoreMesh(
    core_axis_name="c", subcore_axis_name="s", num_cores=1)


def _lane_shift_up(x):
  perm = lax.min(lax.iota(jnp.int32, L) + 1, jnp.full((L,), L - 1, jnp.int32))
  return lax.gather(
      x, perm[:, None],
      lax.GatherDimensionNumbers(
          offset_dims=(), collapsed_slice_dims=(0,), start_index_map=(0,)),
      (1,), mode=lax.GatherScatterMode.PROMISE_IN_BOUNDS)


@functools.partial(
    pl.kernel,
    out_type=(
        jax.ShapeDtypeStruct((B,), jnp.int32),
        jax.ShapeDtypeStruct((B,), jnp.float32),
    ),
    mesh=_mesh,
    scratch_types=[
        pltpu.VMEM((NE,), jnp.int32),
        pltpu.VMEM((B,), jnp.int32),
        pltpu.VMEM((128,), jnp.int32),
        pltpu.VMEM((128,), jnp.float32),
    ],
    compiler_params=pltpu.CompilerParams(needs_layout_passes=False),
)
def _prep(idx_hbm, u_hbm, src_hbm, ub_hbm, w_ref, idxb, idxc, uc):
  w = lax.axis_index("s")
  iota = lax.iota(jnp.int32, L)

  @pl.when(w == 0)
  def _winner():
    pltpu.sync_copy(idx_hbm, idxb)

    def scan_step(k, carry):
      iv = idxb[pl.ds(k * L, L)]
      gi = k * L + iota
      comb = iv * B + gi
      cs = lax.sort(comb)
      ivs = lax.shift_right_arithmetic(cs, 14)
      gis = lax.bitwise_and(cs, jnp.full((L,), B - 1, jnp.int32))
      nxt = _lane_shift_up(ivs)
      m = (ivs != nxt) | (iota == L - 1)
      plsc.store_scatter(w_ref, [ivs], gis, mask=m)
      return carry

    lax.fori_loop(0, B // L, scan_step, 0)

    def src_step(k, carry):
      iv = idxb[pl.ds(k * L, L)]
      idxb[pl.ds(k * L, L)] = plsc.load_gather(w_ref, [iv])
      return carry

    lax.fori_loop(0, B // L, src_step, 0)
    pltpu.sync_copy(idxb, src_hbm)

  @pl.when(w > 0)
  def _ugather():
    def chunk(t, carry):
      cid = (w - 1) * 9 + t

      @pl.when(cid < B // 128)
      def _():
        off = cid * 128
        pltpu.sync_copy(idx_hbm.at[pl.ds(off, 128)], idxc)
        pltpu.sync_copy(u_hbm.at[idxc], uc)
        pltpu.sync_copy(uc, ub_hbm.at[pl.ds(off, 128)])

      return carry

    lax.fori_loop(0, 9, chunk, 0)


_CR = 248
_NCHUNK = -(-NE // _CR)


@functools.partial(
    pl.kernel,
    out_type=jax.ShapeDtypeStruct((NE, D), jnp.float32),
    mesh=_mesh,
    scratch_types=[
        pltpu.VMEM((_CR, D), jnp.float32),
        pltpu.VMEM((128, D), jnp.float32),
        pltpu.VMEM((128,), jnp.int32),
        pltpu.VMEM((128,), jnp.int32),
    ],
)
def _scatter(prev_hbm, emb_hbm, idx_hbm, src_hbm, out_hbm, cb, rows, srcc, idxc):
  w = lax.axis_index("s")

  def copy_chunk(t, carry):
    cid = w + NW * t

    @pl.when(cid < _NCHUNK)
    def _():
      start = pl.multiple_of(lax.min(cid * _CR, NE - _CR), 8)
      pltpu.sync_copy(prev_hbm.at[pl.ds(start, _CR)], cb)
      pltpu.sync_copy(cb, out_hbm.at[pl.ds(start, _CR)])

    return carry

  lax.fori_loop(0, -(-_NCHUNK // NW), copy_chunk, 0)

  plsc.subcore_barrier()

  per = B // NW
  base = w * per

  def scat_chunk(t, carry):
    off = base + t * 128
    pltpu.sync_copy(src_hbm.at[pl.ds(off, 128)], srcc)
    pltpu.sync_copy(idx_hbm.at[pl.ds(off, 128)], idxc)
    pltpu.sync_copy(emb_hbm.at[srcc], rows)
    pltpu.sync_copy(rows, out_hbm.at[idxc])
    return carry

  lax.fori_loop(0, per // 128, scat_chunk, 0)


_R = 512


def _loss_body(acc_ref, logits_ref, oh_ref, emb_ref, ub_ref, cent_ref,
               l1_ref, l2_ref, l3_ref):
  i = pl.program_id(0)

  zero = jnp.zeros((1, 1), jnp.float32)

  @pl.when(i == 0)
  def _():
    l1_ref[...] = zero
    l2_ref[...] = zero
    l3_ref[...] = zero

  acc = acc_ref[0, 0]
  logits = logits_ref[...]
  oh = oh_ref[...]
  emb = emb_ref[...]
  ub = ub_ref[...]

  bn = jnp.sqrt(jnp.sum(emb * emb, axis=1, keepdims=True))
  safe_bn = jnp.where(bn == 0.0, 1.0, bn)
  emb_n = emb / (safe_bn + EPS)
  cent = cent_ref[...]
  cn = jnp.sqrt(jnp.sum(cent * cent, axis=1, keepdims=True))
  safe_cn = jnp.where(cn == 0.0, 1.0, cn)
  cent_n = cent / (safe_cn + EPS)
  sims = lax.dot_general(emb_n, cent_n, (((1,), (1,)), ((), ())),
                         preferred_element_type=jnp.float32)
  smax = jnp.max(sims, axis=1, keepdims=True)
  sexp = jnp.exp(sims - smax)
  soft = sexp / jnp.sum(sexp, axis=1, keepdims=True)

  ml = logits + acc * ub * oh
  mmax = jnp.max(ml, axis=1, keepdims=True)
  msh = ml - mmax
  lse = jnp.log(jnp.sum(jnp.exp(msh), axis=1, keepdims=True))
  logp = msh - lse
  l1p = jnp.sum(-soft * logp)

  cio = lax.broadcasted_iota(jnp.int32, (_R, NC), 1)
  lmax = jnp.max(logits, axis=1, keepdims=True)
  is_max = logits == lmax
  fi = jnp.min(jnp.where(is_max, cio, NC), axis=1, keepdims=True)
  ph = (cio == fi).astype(jnp.float32)
  term = ph + ub * oh - oh
  l2p = jnp.sum(term * term)

  pmax = lmax
  pexp = jnp.exp(logits - pmax)
  prob = pexp / jnp.sum(pexp, axis=1, keepdims=True)
  p_true = jnp.clip(jnp.sum(prob * oh, axis=1, keepdims=True), EPS, 1.0 - EPS)
  u3 = jnp.clip(ub, EPS, 1.0 - EPS)
  u_t = jnp.clip(jax.nn.sigmoid(-jnp.log(u3)), EPS, 1.0 - EPS)
  dkl = (p_true * (jnp.log(p_true) - jnp.log(u_t))
         + (1.0 - p_true) * (jnp.log1p(-p_true) - jnp.log1p(-u_t)))
  l3p = jnp.sum(dkl)

  l1_ref[...] += jnp.reshape(l1p, (1, 1))
  l2_ref[...] += jnp.reshape(l2p, (1, 1))
  l3_ref[...] += jnp.reshape(l3p, (1, 1))

  @pl.when(i == (B // _R) - 1)
  def _():
    l1_ref[...] = l1_ref[...] * (1.0 / B)
    l2_ref[...] = l2_ref[...] * (1.0 / (B * NC))
    l3_ref[...] = l3_ref[...] * ((1.0 - acc) / B)


_loss = pl.pallas_call(
    _loss_body,
    grid=(B // _R,),
    in_specs=[
        pl.BlockSpec(memory_space=pltpu.SMEM),
        pl.BlockSpec((_R, NC), lambda i: (i, 0)),
        pl.BlockSpec((_R, NC), lambda i: (i, 0)),
        pl.BlockSpec((_R, D), lambda i: (i, 0)),
        pl.BlockSpec((_R, 1), lambda i: (i, 0)),
        pl.BlockSpec((NC, D), lambda i: (0, 0)),
    ],
    out_specs=(
        pl.BlockSpec((1, 1), lambda i: (0, 0)),
        pl.BlockSpec((1, 1), lambda i: (0, 0)),
        pl.BlockSpec((1, 1), lambda i: (0, 0)),
    ),
    out_shape=(
        jax.ShapeDtypeStruct((1, 1), jnp.float32),
        jax.ShapeDtypeStruct((1, 1), jnp.float32),
        jax.ShapeDtypeStruct((1, 1), jnp.float32),
    ),
)


def kernel(batch_original_indices, gnn_logits_batch, true_labels_batch_one_hot,
           gnn_embeddings_batch, batch_iter_num, current_epoch,
           atrain_overall_accuracy, u, prev_gnn_embeddings, class_centroids):
  del batch_iter_num, current_epoch
  idx = batch_original_indices.astype(jnp.int32)
  u1 = jnp.reshape(u, (NE,))
  src, ub = _prep(idx, u1)
  out = _scatter(prev_gnn_embeddings, gnn_embeddings_batch, idx, src)
  acc = jnp.reshape(atrain_overall_accuracy.astype(jnp.float32), (1, 1))
  l1, l2, l3 = _loss(acc, gnn_logits_batch, true_labels_batch_one_hot,
                     gnn_embeddings_batch, jnp.reshape(ub, (B, 1)),
                     class_centroids)
  return (jnp.reshape(l1, ()), jnp.reshape(l2, ()), jnp.reshape(l3, ()), out)


# speedup vs baseline: 2.0091x; 1.2704x over previous
import functools

import jax
import jax.numpy as jnp
from jax import lax
from jax.experimental import pallas as pl
from jax.experimental.pallas import tpu as pltpu
from jax.experimental.pallas import tpu_sc as plsc

EPS = 1e-07
NE = 100000
NC = 100
D = 256
B = 16384
L = 16
NW = 16

_mesh = plsc.VectorSubcoreMesh(
    core_axis_name="c", subcore_axis_name="s", num_cores=1)


def _lane_shift_up(x):
  perm = lax.min(lax.iota(jnp.int32, L) + 1, jnp.full((L,), L - 1, jnp.int32))
  return lax.gather(
      x, perm[:, None],
      lax.GatherDimensionNumbers(
          offset_dims=(), collapsed_slice_dims=(0,), start_index_map=(0,)),
      (1,), mode=lax.GatherScatterMode.PROMISE_IN_BOUNDS)


@functools.partial(
    pl.kernel,
    out_type=(
        jax.ShapeDtypeStruct((B,), jnp.int32),
        jax.ShapeDtypeStruct((B,), jnp.float32),
    ),
    mesh=_mesh,
    scratch_types=[
        pltpu.VMEM((NE,), jnp.int32),
        pltpu.VMEM((B,), jnp.int32),
        pltpu.VMEM((128,), jnp.int32),
        pltpu.VMEM((128,), jnp.float32),
    ],
    compiler_params=pltpu.CompilerParams(needs_layout_passes=False),
)
def _prep(idx_hbm, u_hbm, src_hbm, ub_hbm, w_ref, idxb, idxc, uc):
  w = lax.axis_index("s")
  iota = lax.iota(jnp.int32, L)

  @pl.when(w == 0)
  def _winner():
    pltpu.sync_copy(idx_hbm, idxb)

    def scan_step(k, carry):
      iv = idxb[pl.ds(k * L, L)]
      gi = k * L + iota
      comb = iv * B + gi
      cs = lax.sort(comb)
      ivs = lax.shift_right_arithmetic(cs, 14)
      gis = lax.bitwise_and(cs, jnp.full((L,), B - 1, jnp.int32))
      nxt = _lane_shift_up(ivs)
      m = (ivs != nxt) | (iota == L - 1)
      plsc.store_scatter(w_ref, [ivs], gis, mask=m)
      return carry

    lax.fori_loop(0, B // L, scan_step, 0)

    def src_step(k, carry):
      iv = idxb[pl.ds(k * L, L)]
      idxb[pl.ds(k * L, L)] = plsc.load_gather(w_ref, [iv])
      return carry

    lax.fori_loop(0, B // L, src_step, 0)
    pltpu.sync_copy(idxb, src_hbm)

  @pl.when(w > 0)
  def _ugather():
    def chunk(t, carry):
      cid = (w - 1) * 9 + t

      @pl.when(cid < B // 128)
      def _():
        off = cid * 128
        pltpu.sync_copy(idx_hbm.at[pl.ds(off, 128)], idxc)
        pltpu.sync_copy(u_hbm.at[idxc], uc)
        pltpu.sync_copy(uc, ub_hbm.at[pl.ds(off, 128)])

      return carry

    lax.fori_loop(0, 9, chunk, 0)


_CR = 120
_NCHUNK = -(-NE // _CR)
_CPW = -(-_NCHUNK // NW)


@functools.partial(
    pl.kernel,
    out_type=jax.ShapeDtypeStruct((NE, D), jnp.float32),
    mesh=_mesh,
    scratch_types=[
        pltpu.VMEM((2, _CR, D), jnp.float32),
        pltpu.VMEM((2, 128, D), jnp.float32),
        pltpu.VMEM((2, 128), jnp.int32),
        pltpu.VMEM((2, 128), jnp.int32),
        pltpu.SemaphoreType.DMA((2,)),
        pltpu.SemaphoreType.DMA((2,)),
        pltpu.SemaphoreType.DMA((2,)),
        pltpu.SemaphoreType.DMA((2,)),
        pltpu.SemaphoreType.DMA((2,)),
        pltpu.SemaphoreType.DMA((2,)),
    ],
)
def _scatter(prev_hbm, emb_hbm, idx_hbm, src_hbm, out_hbm, cb, rows, srcc,
             idxc, rsem, wsem, gsem, ssem, slsem, ilsem):
  w = lax.axis_index("s")

  def chunk_start(t):
    cid = w + NW * t
    return pl.multiple_of(lax.min(cid * _CR, NE - _CR), 8)

  def start_read(t):
    b = lax.rem(t, 2)
    pltpu.make_async_copy(
        prev_hbm.at[pl.ds(chunk_start(t), _CR)], cb.at[b], rsem.at[b]).start()

  def wait_read(t):
    b = lax.rem(t, 2)
    pltpu.make_async_copy(
        prev_hbm.at[pl.ds(chunk_start(t), _CR)], cb.at[b], rsem.at[b]).wait()

  def start_write(t):
    b = lax.rem(t, 2)
    pltpu.make_async_copy(
        cb.at[b], out_hbm.at[pl.ds(chunk_start(t), _CR)], wsem.at[b]).start()

  def wait_write(t):
    b = lax.rem(t, 2)
    pltpu.make_async_copy(
        cb.at[b], out_hbm.at[pl.ds(chunk_start(t), _CR)], wsem.at[b]).wait()

  start_read(0)

  def copy_body(t, carry):
    @pl.when(t + 1 < _CPW)
    def _():
      start_read(t + 1)

    wait_read(t)
    start_write(t)
    wait_write(t)
    return carry

  lax.fori_loop(0, _CPW, copy_body, 0)

  plsc.subcore_barrier()

  per = B // NW
  base = w * per
  ncs = per // 128

  def load_meta(t):
    b = lax.rem(t, 2)
    off = base + t * 128
    pltpu.make_async_copy(
        src_hbm.at[pl.ds(off, 128)], srcc.at[b], slsem.at[b]).start()
    pltpu.make_async_copy(
        idx_hbm.at[pl.ds(off, 128)], idxc.at[b], ilsem.at[b]).start()

  def wait_meta(t):
    b = lax.rem(t, 2)
    off = base + t * 128
    pltpu.make_async_copy(
        src_hbm.at[pl.ds(off, 128)], srcc.at[b], slsem.at[b]).wait()
    pltpu.make_async_copy(
        idx_hbm.at[pl.ds(off, 128)], idxc.at[b], ilsem.at[b]).wait()

  def wait_scat(t):
    b = lax.rem(t, 2)
    pltpu.make_async_copy(rows.at[b], out_hbm.at[idxc.at[b]], ssem.at[b]).wait()

  load_meta(0)

  def scat_body(t, carry):
    b = lax.rem(t, 2)

    @pl.when(t + 1 < ncs)
    def _():
      load_meta(t + 1)

    wait_meta(t)
    pltpu.make_async_copy(emb_hbm.at[srcc.at[b]], rows.at[b], gsem.at[b]).start()

    @pl.when(t >= 2)
    def _():
      wait_scat(t - 2)

    pltpu.make_async_copy(emb_hbm.at[srcc.at[b]], rows.at[b], gsem.at[b]).wait()
    pltpu.make_async_copy(rows.at[b], out_hbm.at[idxc.at[b]], ssem.at[b]).start()
    return carry

  lax.fori_loop(0, ncs, scat_body, 0)
  wait_scat(ncs - 2)
  wait_scat(ncs - 1)


_R = 512


def _loss_body(acc_ref, logits_ref, oh_ref, emb_ref, ub_ref, cent_ref,
               l1_ref, l2_ref, l3_ref):
  i = pl.program_id(0)

  zero = jnp.zeros((1, 1), jnp.float32)

  @pl.when(i == 0)
  def _():
    l1_ref[...] = zero
    l2_ref[...] = zero
    l3_ref[...] = zero

  acc = acc_ref[0, 0]
  logits = logits_ref[...]
  oh = oh_ref[...]
  emb = emb_ref[...]
  ub = ub_ref[...]

  bn = jnp.sqrt(jnp.sum(emb * emb, axis=1, keepdims=True))
  safe_bn = jnp.where(bn == 0.0, 1.0, bn)
  emb_n = emb / (safe_bn + EPS)
  cent = cent_ref[...]
  cn = jnp.sqrt(jnp.sum(cent * cent, axis=1, keepdims=True))
  safe_cn = jnp.where(cn == 0.0, 1.0, cn)
  cent_n = cent / (safe_cn + EPS)
  sims = lax.dot_general(emb_n, cent_n, (((1,), (1,)), ((), ())),
                         preferred_element_type=jnp.float32)
  smax = jnp.max(sims, axis=1, keepdims=True)
  sexp = jnp.exp(sims - smax)
  soft = sexp / jnp.sum(sexp, axis=1, keepdims=True)

  ml = logits + acc * ub * oh
  mmax = jnp.max(ml, axis=1, keepdims=True)
  msh = ml - mmax
  lse = jnp.log(jnp.sum(jnp.exp(msh), axis=1, keepdims=True))
  logp = msh - lse
  l1p = jnp.sum(-soft * logp)

  cio = lax.broadcasted_iota(jnp.int32, (_R, NC), 1)
  lmax = jnp.max(logits, axis=1, keepdims=True)
  is_max = logits == lmax
  fi = jnp.min(jnp.where(is_max, cio, NC), axis=1, keepdims=True)
  ph = (cio == fi).astype(jnp.float32)
  term = ph + ub * oh - oh
  l2p = jnp.sum(term * term)

  pmax = lmax
  pexp = jnp.exp(logits - pmax)
  prob = pexp / jnp.sum(pexp, axis=1, keepdims=True)
  p_true = jnp.clip(jnp.sum(prob * oh, axis=1, keepdims=True), EPS, 1.0 - EPS)
  u3 = jnp.clip(ub, EPS, 1.0 - EPS)
  u_t = jnp.clip(jax.nn.sigmoid(-jnp.log(u3)), EPS, 1.0 - EPS)
  dkl = (p_true * (jnp.log(p_true) - jnp.log(u_t))
         + (1.0 - p_true) * (jnp.log1p(-p_true) - jnp.log1p(-u_t)))
  l3p = jnp.sum(dkl)

  l1_ref[...] += jnp.reshape(l1p, (1, 1))
  l2_ref[...] += jnp.reshape(l2p, (1, 1))
  l3_ref[...] += jnp.reshape(l3p, (1, 1))

  @pl.when(i == (B // _R) - 1)
  def _():
    l1_ref[...] = l1_ref[...] * (1.0 / B)
    l2_ref[...] = l2_ref[...] * (1.0 / (B * NC))
    l3_ref[...] = l3_ref[...] * ((1.0 - acc) / B)


_loss = pl.pallas_call(
    _loss_body,
    grid=(B // _R,),
    in_specs=[
        pl.BlockSpec(memory_space=pltpu.SMEM),
        pl.BlockSpec((_R, NC), lambda i: (i, 0)),
        pl.BlockSpec((_R, NC), lambda i: (i, 0)),
        pl.BlockSpec((_R, D), lambda i: (i, 0)),
        pl.BlockSpec((_R, 1), lambda i: (i, 0)),
        pl.BlockSpec((NC, D), lambda i: (0, 0)),
    ],
    out_specs=(
        pl.BlockSpec((1, 1), lambda i: (0, 0)),
        pl.BlockSpec((1, 1), lambda i: (0, 0)),
        pl.BlockSpec((1, 1), lambda i: (0, 0)),
    ),
    out_shape=(
        jax.ShapeDtypeStruct((1, 1), jnp.float32),
        jax.ShapeDtypeStruct((1, 1), jnp.float32),
        jax.ShapeDtypeStruct((1, 1), jnp.float32),
    ),
)


def kernel(batch_original_indices, gnn_logits_batch, true_labels_batch_one_hot,
           gnn_embeddings_batch, batch_iter_num, current_epoch,
           atrain_overall_accuracy, u, prev_gnn_embeddings, class_centroids):
  del batch_iter_num, current_epoch
  idx = batch_original_indices.astype(jnp.int32)
  u1 = jnp.reshape(u, (NE,))
  src, ub = _prep(idx, u1)
  out = _scatter(prev_gnn_embeddings, gnn_embeddings_batch, idx, src)
  acc = jnp.reshape(atrain_overall_accuracy.astype(jnp.float32), (1, 1))
  l1, l2, l3 = _loss(acc, gnn_logits_batch, true_labels_batch_one_hot,
                     gnn_embeddings_batch, jnp.reshape(ub, (B, 1)),
                     class_centroids)
  return (jnp.reshape(l1, ()), jnp.reshape(l2, ()), jnp.reshape(l3, ()), out)


# speedup vs baseline: 2.3701x; 1.1797x over previous
import functools

import jax
import jax.numpy as jnp
from jax import lax
from jax.experimental import pallas as pl
from jax.experimental.pallas import tpu as pltpu
from jax.experimental.pallas import tpu_sc as plsc

EPS = 1e-07
NE = 100000
NC = 100
D = 256
B = 16384
L = 16
NCORE = 2
NSUB = 16
HALF = NE // NCORE

_mesh = plsc.VectorSubcoreMesh(
    core_axis_name="c", subcore_axis_name="s", num_cores=NCORE)


def _lane_perm(x, perm):
  return lax.gather(
      x, perm[:, None],
      lax.GatherDimensionNumbers(
          offset_dims=(), collapsed_slice_dims=(0,), start_index_map=(0,)),
      (1,), mode=lax.GatherScatterMode.PROMISE_IN_BOUNDS)


def _lane_shift_up(x):
  perm = lax.min(lax.iota(jnp.int32, L) + 1, jnp.full((L,), L - 1, jnp.int32))
  return _lane_perm(x, perm)


@functools.partial(
    pl.kernel,
    out_type=jax.ShapeDtypeStruct((B,), jnp.float32),
    mesh=_mesh,
    scratch_types=[
        pltpu.VMEM((128,), jnp.int32),
        pltpu.VMEM((128,), jnp.float32),
    ],
)
def _ugather(idx_hbm, u_hbm, ub_hbm, idxc, uc):
  c = lax.axis_index("c")
  s = lax.axis_index("s")

  @pl.when(s > 0)
  def _():
    gw = (s - 1) * NCORE + c

    def chunk(t, carry):
      cid = gw * 5 + t

      @pl.when(cid < B // 128)
      def _():
        off = cid * 128
        pltpu.sync_copy(idx_hbm.at[pl.ds(off, 128)], idxc)
        pltpu.sync_copy(u_hbm.at[idxc], uc)
        pltpu.sync_copy(uc, ub_hbm.at[pl.ds(off, 128)])

      return carry

    lax.fori_loop(0, 5, chunk, 0)


_CR = 120
_NCH = -(-HALF // _CR)
_NCOPY = NSUB - 1
_CPW = -(-_NCH // _NCOPY)
_PB = B // NSUB
_KCH = _PB // 128 + 1


@functools.partial(
    pl.kernel,
    out_type=jax.ShapeDtypeStruct((NE, D), jnp.float32),
    mesh=_mesh,
    scratch_types=[
        pltpu.VMEM_SHARED((B,), jnp.int32),
        pltpu.VMEM_SHARED((NSUB, _KCH * 128), jnp.int32),
        pltpu.SemaphoreType.DMA((2,)),
        pltpu.SemaphoreType.DMA((2,)),
        pltpu.SemaphoreType.DMA((2,)),
        pltpu.SemaphoreType.DMA((2,)),
    ],
    compiler_params=pltpu.CompilerParams(needs_layout_passes=False),
)
def _scatter(prev_hbm, emb_hbm, idx_hbm, out_hbm, src_sh, ksp, rsem, wsem,
             gsem, ssem):
  c = lax.axis_index("c")
  s = lax.axis_index("s")
  lo = c * HALF
  iota = lax.iota(jnp.int32, L)

  @pl.when(s == 0)
  def _scan():
    def scan_fn(w_ref, idxb):
      pltpu.sync_copy(idx_hbm, idxb)

      def scan_step(k, carry):
        iv = idxb[pl.ds(k * L, L)]
        gi = k * L + iota
        comb = iv * B + gi
        cs = lax.sort(comb)
        ivs = lax.shift_right_arithmetic(cs, 14)
        gis = lax.bitwise_and(cs, jnp.full((L,), B - 1, jnp.int32))
        nxt = _lane_shift_up(ivs)
        own = (ivs >= lo) & (ivs < lo + HALF)
        m = ((ivs != nxt) | (iota == L - 1)) & own
        wi = jnp.clip(ivs - lo, 0, HALF - 1)
        plsc.store_scatter(w_ref, [wi], gis, mask=m)
        return carry

      lax.fori_loop(0, B // L, scan_step, 0)

      def src_step(k, carry):
        iv = idxb[pl.ds(k * L, L)]
        wi = jnp.clip(iv - lo, 0, HALF - 1)
        idxb[pl.ds(k * L, L)] = plsc.load_gather(w_ref, [wi])
        return carry

      lax.fori_loop(0, B // L, src_step, 0)
      pltpu.sync_copy(idxb, src_sh)

    pl.run_scoped(
        scan_fn,
        pltpu.VMEM((HALF,), jnp.int32),
        pltpu.VMEM((B,), jnp.int32),
    )

  @pl.when(s > 0)
  def _copy():
    def copy_fn(cb):
      def chunk_start(t):
        cid = (s - 1) + _NCOPY * t
        return pl.multiple_of(
            lo + lax.min(cid * _CR, HALF - _CR), 8)

      def start_read(t):
        b = lax.rem(t, 2)
        pltpu.make_async_copy(
            prev_hbm.at[pl.ds(chunk_start(t), _CR)], cb.at[b],
            rsem.at[b]).start()

      def wait_read(t):
        b = lax.rem(t, 2)
        pltpu.make_async_copy(
            prev_hbm.at[pl.ds(chunk_start(t), _CR)], cb.at[b],
            rsem.at[b]).wait()

      def start_write(t):
        b = lax.rem(t, 2)
        pltpu.make_async_copy(
            cb.at[b], out_hbm.at[pl.ds(chunk_start(t), _CR)],
            wsem.at[b]).start()

      def wait_write(t):
        b = lax.rem(t, 2)
        pltpu.make_async_copy(
            cb.at[b], out_hbm.at[pl.ds(chunk_start(t), _CR)],
            wsem.at[b]).wait()

      start_read(0)

      def copy_body(t, carry):
        @pl.when(t + 1 < _CPW)
        def _():
          start_read(t + 1)

        wait_read(t)
        start_write(t)
        wait_write(t)
        return carry

      lax.fori_loop(0, _CPW, copy_body, 0)

    pl.run_scoped(copy_fn, pltpu.VMEM((2, _CR, D), jnp.float32))

  plsc.subcore_barrier()

  def pb_fn(idx_sl, src_sl, kli, kls, k2i, rows):
    base = s * _PB
    pltpu.sync_copy(idx_hbm.at[pl.ds(base, _PB)], idx_sl)
    pltpu.sync_copy(src_sh.at[pl.ds(base, _PB)], src_sl)

    def compact(k, off):
      iv = idx_sl[pl.ds(k * L, L)]
      sv = src_sl[pl.ds(k * L, L)]
      m = (iv >= lo) & (iv < lo + HALF)
      plsc.store_compressed(kli.at[pl.ds(off, L)], iv, mask=m)
      plsc.store_compressed(kls.at[pl.ds(off, L)], sv, mask=m)
      return off + jnp.sum(m.astype(jnp.int32))

    n = lax.fori_loop(0, _PB // L, compact, 0)

    @pl.when(n > 0)
    def _():
      zperm = jnp.zeros((L,), jnp.int32)
      k0i = _lane_perm(kli[pl.ds(0, L)], zperm)
      k0s = _lane_perm(kls[pl.ds(0, L)], zperm)

      def pad(j, carry):
        pos = j * L + iota
        m = pos >= n
        cur_i = kli[pl.ds(j * L, L)]
        cur_s = kls[pl.ds(j * L, L)]
        kli[pl.ds(j * L, L)] = jnp.where(m, k0i, cur_i)
        kls[pl.ds(j * L, L)] = jnp.where(m, k0s, cur_s)
        return carry

      lax.fori_loop(0, _KCH * 8, pad, 0)

      pltpu.sync_copy(kli, ksp.at[s])
      for t in range(_KCH):
        pltpu.sync_copy(ksp.at[s, pl.ds(t * 128, 128)], k2i.at[t])

      for t in range(_KCH):
        @pl.when(t * 128 < n)
        def _(t=t):
          b = t % 2
          if t >= 2:
            pltpu.make_async_copy(
                rows.at[b], out_hbm.at[k2i.at[t - 2]], ssem.at[b]).wait()
          gat = pltpu.make_async_copy(
              emb_hbm.at[kls.at[pl.ds(t * 128, 128)]], rows.at[b], gsem.at[b])
          gat.start()
          gat.wait()
          pltpu.make_async_copy(
              rows.at[b], out_hbm.at[k2i.at[t]], ssem.at[b]).start()

      for t in range(_KCH):
        @pl.when((t * 128 < n) & ((t + 2) * 128 >= n))
        def _(t=t):
          b = t % 2
          pltpu.make_async_copy(
              rows.at[b], out_hbm.at[k2i.at[t]], ssem.at[b]).wait()

  pl.run_scoped(
      pb_fn,
      pltpu.VMEM((_PB,), jnp.int32),
      pltpu.VMEM((_PB,), jnp.int32),
      pltpu.VMEM((_KCH * 128,), jnp.int32),
      pltpu.VMEM((_KCH * 128,), jnp.int32),
      pltpu.VMEM((_KCH, 128), jnp.int32),
      pltpu.VMEM((2, 128, D), jnp.float32),
  )


_R = 512


def _loss_body(acc_ref, logits_ref, oh_ref, emb_ref, ub_ref, cent_ref,
               l1_ref, l2_ref, l3_ref):
  i = pl.program_id(0)

  zero = jnp.zeros((1, 1), jnp.float32)

  @pl.when(i == 0)
  def _():
    l1_ref[...] = zero
    l2_ref[...] = zero
    l3_ref[...] = zero

  acc = acc_ref[0, 0]
  logits = logits_ref[...]
  oh = oh_ref[...]
  emb = emb_ref[...]
  ub = ub_ref[...]

  bn = jnp.sqrt(jnp.sum(emb * emb, axis=1, keepdims=True))
  safe_bn = jnp.where(bn == 0.0, 1.0, bn)
  emb_n = emb / (safe_bn + EPS)
  cent = cent_ref[...]
  cn = jnp.sqrt(jnp.sum(cent * cent, axis=1, keepdims=True))
  safe_cn = jnp.where(cn == 0.0, 1.0, cn)
  cent_n = cent / (safe_cn + EPS)
  sims = lax.dot_general(emb_n, cent_n, (((1,), (1,)), ((), ())),
                         preferred_element_type=jnp.float32)
  smax = jnp.max(sims, axis=1, keepdims=True)
  sexp = jnp.exp(sims - smax)
  soft = sexp / jnp.sum(sexp, axis=1, keepdims=True)

  ml = logits + acc * ub * oh
  mmax = jnp.max(ml, axis=1, keepdims=True)
  msh = ml - mmax
  lse = jnp.log(jnp.sum(jnp.exp(msh), axis=1, keepdims=True))
  logp = msh - lse
  l1p = jnp.sum(-soft * logp)

  cio = lax.broadcasted_iota(jnp.int32, (_R, NC), 1)
  lmax = jnp.max(logits, axis=1, keepdims=True)
  is_max = logits == lmax
  fi = jnp.min(jnp.where(is_max, cio, NC), axis=1, keepdims=True)
  ph = (cio == fi).astype(jnp.float32)
  term = ph + ub * oh - oh
  l2p = jnp.sum(term * term)

  pexp = jnp.exp(logits - lmax)
  prob = pexp / jnp.sum(pexp, axis=1, keepdims=True)
  p_true = jnp.clip(jnp.sum(prob * oh, axis=1, keepdims=True), EPS, 1.0 - EPS)
  u3 = jnp.clip(ub, EPS, 1.0 - EPS)
  u_t = jnp.clip(jax.nn.sigmoid(-jnp.log(u3)), EPS, 1.0 - EPS)
  dkl = (p_true * (jnp.log(p_true) - jnp.log(u_t))
         + (1.0 - p_true) * (jnp.log1p(-p_true) - jnp.log1p(-u_t)))
  l3p = jnp.sum(dkl)

  l1_ref[...] += jnp.reshape(l1p, (1, 1))
  l2_ref[...] += jnp.reshape(l2p, (1, 1))
  l3_ref[...] += jnp.reshape(l3p, (1, 1))

  @pl.when(i == (B // _R) - 1)
  def _():
    l1_ref[...] = l1_ref[...] * (1.0 / B)
    l2_ref[...] = l2_ref[...] * (1.0 / (B * NC))
    l3_ref[...] = l3_ref[...] * ((1.0 - acc) / B)


_loss = pl.pallas_call(
    _loss_body,
    grid=(B // _R,),
    in_specs=[
        pl.BlockSpec(memory_space=pltpu.SMEM),
        pl.BlockSpec((_R, NC), lambda i: (i, 0)),
        pl.BlockSpec((_R, NC), lambda i: (i, 0)),
        pl.BlockSpec((_R, D), lambda i: (i, 0)),
        pl.BlockSpec((_R, 1), lambda i: (i, 0)),
        pl.BlockSpec((NC, D), lambda i: (0, 0)),
    ],
    out_specs=(
        pl.BlockSpec((1, 1), lambda i: (0, 0)),
        pl.BlockSpec((1, 1), lambda i: (0, 0)),
        pl.BlockSpec((1, 1), lambda i: (0, 0)),
    ),
    out_shape=(
        jax.ShapeDtypeStruct((1, 1), jnp.float32),
        jax.ShapeDtypeStruct((1, 1), jnp.float32),
        jax.ShapeDtypeStruct((1, 1), jnp.float32),
    ),
)


def kernel(batch_original_indices, gnn_logits_batch, true_labels_batch_one_hot,
           gnn_embeddings_batch, batch_iter_num, current_epoch,
           atrain_overall_accuracy, u, prev_gnn_embeddings, class_centroids):
  del batch_iter_num, current_epoch
  idx = batch_original_indices.astype(jnp.int32)
  u1 = jnp.reshape(u, (NE,))
  ub = _ugather(idx, u1)
  out = _scatter(prev_gnn_embeddings, gnn_embeddings_batch, idx)
  acc = jnp.reshape(atrain_overall_accuracy.astype(jnp.float32), (1, 1))
  l1, l2, l3 = _loss(acc, gnn_logits_batch, true_labels_batch_one_hot,
                     gnn_embeddings_batch, jnp.reshape(ub, (B, 1)),
                     class_centroids)
  return (jnp.reshape(l1, ()), jnp.reshape(l2, ()), jnp.reshape(l3, ()), out)
